# baseline (speedup 1.0000x reference)
import functools

import jax
import jax.numpy as jnp
from jax import lax
from jax.experimental import pallas as pl
from jax.experimental.pallas import tpu as pltpu
from jax.experimental.pallas import tpu_sc as plsc

NC = 2
NS = 16
NW = NC * NS
L = 16


def _mesh():
    return plsc.VectorSubcoreMesh(
        core_axis_name="c", subcore_axis_name="s", num_cores=NC, num_subcores=NS
    )


def _sc_params():
    return pltpu.CompilerParams(needs_layout_passes=False)


def _wid():
    return lax.axis_index("s") * NC + lax.axis_index("c")


def _make_deg_kernel(n, e):
    ept = e // NW
    ch = 2000

    @functools.partial(
        pl.kernel,
        out_type=jax.ShapeDtypeStruct((NW * n,), jnp.float32),
        mesh=_mesh(),
        compiler_params=_sc_params(),
        scratch_types=[
            pltpu.VMEM((ch,), jnp.int32),
            pltpu.VMEM((ch,), jnp.float32),
            pltpu.VMEM((n,), jnp.float32),
        ],
    )
    def deg_kernel(dst_hbm, w_hbm, zero_hbm, out_hbm, dst_v, w_v, acc):
        wid = _wid()
        base = wid * ept
        pltpu.sync_copy(zero_hbm.at[pl.ds(0, n)], acc)

        def chunk(g, _):
            off = base + g * ch
            pltpu.sync_copy(dst_hbm.at[pl.ds(off, ch)], dst_v)
            pltpu.sync_copy(w_hbm.at[pl.ds(off, ch)], w_v)

            def body(j, _):
                d = dst_v[pl.ds(j * L, L)]
                v = w_v[pl.ds(j * L, L)]
                plsc.addupdate_scatter(acc, [d], v)
                return 0

            lax.fori_loop(0, ch // L, body, 0)
            return 0

        lax.fori_loop(0, ept // ch, chunk, 0)
        pltpu.sync_copy(acc, out_hbm.at[pl.ds(wid * n, n)])

    return deg_kernel


def _make_norm_kernel(n, e):
    ept = e // NW
    ch = 2000

    @functools.partial(
        pl.kernel,
        out_type=(jax.ShapeDtypeStruct((e,), jnp.float32),
                  jax.ShapeDtypeStruct((e,), jnp.int32)),
        mesh=_mesh(),
        compiler_params=_sc_params(),
        scratch_types=[
            pltpu.VMEM((n,), jnp.float32),
            pltpu.VMEM((ch,), jnp.int32),
            pltpu.VMEM((ch,), jnp.int32),
            pltpu.VMEM((ch,), jnp.float32),
            pltpu.VMEM((ch,), jnp.float32),
            pltpu.VMEM((ch,), jnp.int32),
        ],
    )
    def norm_kernel(src_hbm, dst_hbm, w_hbm, dis_hbm, out_hbm, pk_hbm,
                    dis_v, src_v, dst_v, w_v, nrm_v, pk_v):
        wid = _wid()
        base = wid * ept
        pltpu.sync_copy(dis_hbm, dis_v)

        def chunk(g, _):
            off = base + g * ch
            pltpu.sync_copy(src_hbm.at[pl.ds(off, ch)], src_v)
            pltpu.sync_copy(dst_hbm.at[pl.ds(off, ch)], dst_v)
            pltpu.sync_copy(w_hbm.at[pl.ds(off, ch)], w_v)

            def body(j, _):
                s = src_v[pl.ds(j * L, L)]
                d = dst_v[pl.ds(j * L, L)]
                wv = w_v[pl.ds(j * L, L)]
                ds_ = plsc.load_gather(dis_v, [s])
                dd_ = plsc.load_gather(dis_v, [d])
                nrm_v[pl.ds(j * L, L)] = ds_ * wv * dd_
                pk_v[pl.ds(j * L, L)] = lax.bitwise_or(s, lax.shift_left(d, 16))
                return 0

            lax.fori_loop(0, ch // L, body, 0)
            pltpu.sync_copy(nrm_v, out_hbm.at[pl.ds(off, ch)])
            pltpu.sync_copy(pk_v, pk_hbm.at[pl.ds(off, ch)])
            return 0

        lax.fori_loop(0, ept // ch, chunk, 0)

    return norm_kernel


def _make_agg_kernel(n, e, f):
    fpt = f // NW
    npp = fpt // 2
    fh = f // 2
    ch = 8000
    nch = e // ch
    assert e % ch == 0 and nch % 2 == 0 and ch % L == 0

    @functools.partial(
        pl.kernel,
        out_type=jax.ShapeDtypeStruct((f * n,), jnp.float32),
        mesh=_mesh(),
        compiler_params=_sc_params(),
        scratch_types=[
            pltpu.VMEM((npp * n,), jnp.int32),
            pltpu.VMEM((fpt * n,), jnp.float32),
            pltpu.VMEM((ch,), jnp.int32),
            pltpu.VMEM((ch,), jnp.int32),
            pltpu.VMEM((ch,), jnp.float32),
            pltpu.VMEM((ch,), jnp.float32),
            pltpu.SemaphoreType.DMA,
            pltpu.SemaphoreType.DMA,
        ],
    )
    def agg_kernel(hP_hbm, pk_hbm, nrm_hbm, zero_hbm, out_hbm,
                   h_v, acc, pk0, pk1, nrm0, nrm1, sem0, sem1):
        wid = _wid()
        pltpu.sync_copy(hP_hbm.at[pl.ds(wid * (npp * n), npp * n)], h_v)
        pltpu.sync_copy(zero_hbm, acc)
        bufs = ((pk0, nrm0, sem0), (pk1, nrm1, sem1))

        def start(g, p):
            off = g * ch
            pv, nv, sem = bufs[p]
            pltpu.async_copy(pk_hbm.at[pl.ds(off, ch)], pv, sem)
            pltpu.async_copy(nrm_hbm.at[pl.ds(off, ch)], nv, sem)

        def wait(g, p):
            off = g * ch
            pv, nv, sem = bufs[p]
            pltpu.make_async_copy(pk_hbm.at[pl.ds(off, ch)], pv, sem).wait()
            pltpu.make_async_copy(nrm_hbm.at[pl.ds(off, ch)], nv, sem).wait()

        def compute(p):
            pv, nv, _ = bufs[p]

            @plsc.parallel_loop(0, ch, step=L, unroll=8)
            def body(i):
                pk = pv[pl.ds(i, L)]
                nrm = nv[pl.ds(i, L)]
                s = lax.bitwise_and(pk, 0xFFFF)
                d = lax.shift_right_logical(pk, 16)
                for pp in range(npp):
                    si = s + (pp * n) if pp else s
                    w = plsc.load_gather(h_v, [si])
                    ab = plsc.unpack(plsc.bitcast(w, jnp.bfloat16),
                                     format=plsc.PackFormat.INTERLEAVED)
                    lo, hi = ab
                    dlo = d + (pp * n) if pp else d
                    plsc.addupdate_scatter(acc, [dlo], nrm * lo)
                    plsc.addupdate_scatter(acc, [d + ((npp + pp) * n)], nrm * hi)

        start(0, 0)

        def outer(g2, _):
            g0 = g2 * 2
            start(g0 + 1, 1)
            wait(g0, 0)
            compute(0)

            @pl.when(g2 + 1 < nch // 2)
            def _prefetch():
                start(g0 + 2, 0)

            wait(g0 + 1, 1)
            compute(1)
            return 0

        lax.fori_loop(0, nch // 2, outer, 0)
        pltpu.sync_copy(acc.at[pl.ds(0, npp * n)],
                        out_hbm.at[pl.ds((npp * wid) * n, npp * n)])
        pltpu.sync_copy(acc.at[pl.ds(npp * n, npp * n)],
                        out_hbm.at[pl.ds((fh + npp * wid) * n, npp * n)])

    return agg_kernel


def _make_agg2_kernel(n, e, f):
    nchunks = NW // f
    ept = e // nchunks
    ch = 4000

    @functools.partial(
        pl.kernel,
        out_type=jax.ShapeDtypeStruct((NW * n,), jnp.float32),
        mesh=_mesh(),
        compiler_params=_sc_params(),
        scratch_types=[
            pltpu.VMEM((n,), jnp.float32),
            pltpu.VMEM((n,), jnp.float32),
            pltpu.VMEM((ch,), jnp.int32),
            pltpu.VMEM((ch,), jnp.float32),
        ],
    )
    def agg2_kernel(hT_hbm, pk_hbm, nrm_hbm, zero_hbm, out_hbm,
                    h_v, acc, pk_v, nrm_v):
        wid = _wid()
        ff = lax.rem(wid, f)
        cc = lax.div(wid, f)
        base = cc * ept
        pltpu.sync_copy(hT_hbm.at[pl.ds(ff * n, n)], h_v)
        pltpu.sync_copy(zero_hbm.at[pl.ds(0, n)], acc)

        def chunk(g, _):
            off = base + g * ch
            pltpu.sync_copy(pk_hbm.at[pl.ds(off, ch)], pk_v)
            pltpu.sync_copy(nrm_hbm.at[pl.ds(off, ch)], nrm_v)

            @plsc.parallel_loop(0, ch, step=L, unroll=8)
            def body(i):
                pk = pk_v[pl.ds(i, L)]
                nv = nrm_v[pl.ds(i, L)]
                s = lax.bitwise_and(pk, 0xFFFF)
                d = lax.shift_right_logical(pk, 16)
                hv = plsc.load_gather(h_v, [s])
                plsc.addupdate_scatter(acc, [d], nv * hv)

            return 0

        lax.fori_loop(0, ept // ch, chunk, 0)
        pltpu.sync_copy(acc, out_hbm.at[pl.ds(wid * n, n)])

    return agg2_kernel


def _degfin_body(p_ref, dis_ref, dis2_ref):
    deg = jnp.sum(p_ref[...], axis=0) + 1.0
    dis = jnp.where(deg > 0, lax.rsqrt(jnp.maximum(deg, 1e-12)), 0.0)
    dis_ref[...] = dis
    dis2_ref[...] = dis * dis


def _pack_pairs(hT):
    fh = hT.shape[0] // 2
    hb = hT.astype(jnp.bfloat16)
    u = lax.bitcast_convert_type(hb, jnp.uint16).astype(jnp.uint32)
    w = lax.bitwise_or(u[:fh], lax.shift_left(u[fh:], jnp.uint32(16)))
    return lax.bitcast_convert_type(w, jnp.int32)


def _mm1_body(w_ref, x_ref, o_ref, p_ref):
    hT = lax.dot_general(
        w_ref[...], x_ref[...], (([0], [1]), ([], [])),
        preferred_element_type=jnp.float32)
    o_ref[...] = hT
    p_ref[...] = _pack_pairs(hT)


def _layer_body(agg_ref, h_ref, dis2_ref, b_ref, w_ref, o_ref, p_ref):
    z = jnp.maximum(agg_ref[...] + dis2_ref[...] * h_ref[...] + b_ref[...], 0.0)
    hT = lax.dot_general(
        w_ref[...], z, (([0], [0]), ([], [])),
        preferred_element_type=jnp.float32)
    o_ref[...] = hT
    p_ref[...] = _pack_pairs(hT)


def _layer3_body(agg_ref, h_ref, dis2_ref, b_ref, w3_ref, wlin_ref, o_ref):
    z = jnp.maximum(agg_ref[...] + dis2_ref[...] * h_ref[...] + b_ref[...], 0.0)
    w3p = lax.dot_general(
        w3_ref[...], wlin_ref[...], (([1], [0]), ([], [])),
        preferred_element_type=jnp.float32)
    o_ref[...] = lax.dot_general(
        w3p, z, (([0], [0]), ([], [])),
        preferred_element_type=jnp.float32)


def _final_body(p_ref, h3_ref, dis2_ref, batch_ref, b3_ref, wlin_ref, blin_ref,
                o_ref, *, n, g):
    f = h3_ref.shape[0]
    agg3 = jnp.sum(p_ref[...].reshape(NW // f, f, n), axis=0)
    z3 = agg3 + dis2_ref[...] * h3_ref[...]
    gi = lax.broadcasted_iota(jnp.int32, (g, n), 0)
    m = (gi == batch_ref[...][None, :]).astype(jnp.float32)
    sums = lax.dot_general(m, z3, (([1], [1]), ([], [])),
                           preferred_element_type=jnp.float32)
    counts = lax.dot_general(m, jnp.ones((1, n), jnp.float32),
                             (([1], [1]), ([], [])),
                             preferred_element_type=jnp.float32)
    const = lax.dot_general(b3_ref[...], wlin_ref[...], (([1], [0]), ([], [])),
                            preferred_element_type=jnp.float32)
    o_ref[...] = sums / jnp.maximum(counts, 1.0) + const + blin_ref[...][None, :]


def kernel(x, edge_index, edge_weight, batch, W1, b1, W2, b2, W3, b3, Wlin, blin):
    n, d = x.shape
    e = edge_index.shape[1]
    h = W1.shape[1]
    c = Wlin.shape[1]
    g = 64
    fpt = h // NW

    src = edge_index[0].astype(jnp.int32)
    dst = edge_index[1].astype(jnp.int32)
    zeros_flat = jnp.zeros((fpt * n,), jnp.float32)

    deg_k = _make_deg_kernel(n, e)
    norm_k = _make_norm_kernel(n, e)
    agg_k = _make_agg_kernel(n, e, h)
    agg2_k = _make_agg2_kernel(n, e, c)

    partials = deg_k(dst, edge_weight, zeros_flat).reshape(NW, n)
    dis, dis2 = pl.pallas_call(
        _degfin_body,
        out_shape=(jax.ShapeDtypeStruct((n,), jnp.float32),
                   jax.ShapeDtypeStruct((n,), jnp.float32)),
    )(partials)
    norm, pk = norm_k(src, dst, edge_weight, dis)

    dis2r = dis2[None, :]
    h1T, h1P = pl.pallas_call(
        _mm1_body,
        out_shape=(jax.ShapeDtypeStruct((h, n), jnp.float32),
                   jax.ShapeDtypeStruct((h // 2, n), jnp.int32)))(W1, x)
    a1T = agg_k(h1P.reshape(-1), pk, norm, zeros_flat).reshape(h, n)
    h2T, h2P = pl.pallas_call(
        _layer_body,
        out_shape=(jax.ShapeDtypeStruct((h, n), jnp.float32),
                   jax.ShapeDtypeStruct((h // 2, n), jnp.int32)))(
        a1T, h1T, dis2r, b1[:, None], W2)
    a2T = agg_k(h2P.reshape(-1), pk, norm, zeros_flat).reshape(h, n)
    h3T = pl.pallas_call(
        _layer3_body, out_shape=jax.ShapeDtypeStruct((c, n), jnp.float32))(
        a2T, h2T, dis2r, b2[:, None], W3, Wlin)
    p3 = agg2_k(h3T.reshape(-1), pk, norm, zeros_flat).reshape(NW, n)
    out = pl.pallas_call(
        functools.partial(_final_body, n=n, g=g),
        out_shape=jax.ShapeDtypeStruct((g, c), jnp.float32),
    )(p3, h3T, dis2r, batch.astype(jnp.int32), b3[None, :], Wlin, blin)
    return out


# speedup vs baseline: 1.0004x; 1.0004x over previous
import functools

import jax
import jax.numpy as jnp
from jax import lax
from jax.experimental import pallas as pl
from jax.experimental.pallas import tpu as pltpu
from jax.experimental.pallas import tpu_sc as plsc

NC = 2
NS = 16
NW = NC * NS
L = 16


def _mesh():
    return plsc.VectorSubcoreMesh(
        core_axis_name="c", subcore_axis_name="s", num_cores=NC, num_subcores=NS
    )


def _sc_params():
    return pltpu.CompilerParams(needs_layout_passes=False)


def _wid():
    return lax.axis_index("s") * NC + lax.axis_index("c")


def _make_deg_kernel(n, e):
    ept = e // NW
    ch = 2000

    @functools.partial(
        pl.kernel,
        out_type=jax.ShapeDtypeStruct((NW * n,), jnp.float32),
        mesh=_mesh(),
        compiler_params=_sc_params(),
        scratch_types=[
            pltpu.VMEM((ch,), jnp.int32),
            pltpu.VMEM((ch,), jnp.float32),
            pltpu.VMEM((n,), jnp.float32),
        ],
    )
    def deg_kernel(dst_hbm, w_hbm, zero_hbm, out_hbm, dst_v, w_v, acc):
        wid = _wid()
        base = wid * ept
        pltpu.sync_copy(zero_hbm.at[pl.ds(0, n)], acc)

        def chunk(g, _):
            off = base + g * ch
            pltpu.sync_copy(dst_hbm.at[pl.ds(off, ch)], dst_v)
            pltpu.sync_copy(w_hbm.at[pl.ds(off, ch)], w_v)

            def body(j, _):
                d = dst_v[pl.ds(j * L, L)]
                v = w_v[pl.ds(j * L, L)]
                plsc.addupdate_scatter(acc, [d], v)
                return 0

            lax.fori_loop(0, ch // L, body, 0)
            return 0

        lax.fori_loop(0, ept // ch, chunk, 0)
        pltpu.sync_copy(acc, out_hbm.at[pl.ds(wid * n, n)])

    return deg_kernel


def _make_norm_kernel(n, e):
    ept = e // NW
    ch = 2000

    @functools.partial(
        pl.kernel,
        out_type=(jax.ShapeDtypeStruct((e,), jnp.float32),
                  jax.ShapeDtypeStruct((e,), jnp.int32)),
        mesh=_mesh(),
        compiler_params=_sc_params(),
        scratch_types=[
            pltpu.VMEM((n,), jnp.float32),
            pltpu.VMEM((ch,), jnp.int32),
            pltpu.VMEM((ch,), jnp.int32),
            pltpu.VMEM((ch,), jnp.float32),
            pltpu.VMEM((ch,), jnp.float32),
            pltpu.VMEM((ch,), jnp.int32),
        ],
    )
    def norm_kernel(src_hbm, dst_hbm, w_hbm, dis_hbm, out_hbm, pk_hbm,
                    dis_v, src_v, dst_v, w_v, nrm_v, pk_v):
        wid = _wid()
        base = wid * ept
        pltpu.sync_copy(dis_hbm, dis_v)

        def chunk(g, _):
            off = base + g * ch
            pltpu.sync_copy(src_hbm.at[pl.ds(off, ch)], src_v)
            pltpu.sync_copy(dst_hbm.at[pl.ds(off, ch)], dst_v)
            pltpu.sync_copy(w_hbm.at[pl.ds(off, ch)], w_v)

            def body(j, _):
                s = src_v[pl.ds(j * L, L)]
                d = dst_v[pl.ds(j * L, L)]
                wv = w_v[pl.ds(j * L, L)]
                ds_ = plsc.load_gather(dis_v, [s])
                dd_ = plsc.load_gather(dis_v, [d])
                nrm_v[pl.ds(j * L, L)] = ds_ * wv * dd_
                pk_v[pl.ds(j * L, L)] = lax.bitwise_or(s, lax.shift_left(d, 16))
                return 0

            lax.fori_loop(0, ch // L, body, 0)
            pltpu.sync_copy(nrm_v, out_hbm.at[pl.ds(off, ch)])
            pltpu.sync_copy(pk_v, pk_hbm.at[pl.ds(off, ch)])
            return 0

        lax.fori_loop(0, ept // ch, chunk, 0)

    return norm_kernel


def _make_agg_kernel(n, e, f):
    fpt = f // NW
    npp = fpt // 2
    fh = f // 2
    ch = 8000
    nch = e // ch
    assert e % ch == 0 and nch % 2 == 0 and ch % L == 0

    @functools.partial(
        pl.kernel,
        out_type=jax.ShapeDtypeStruct((f * n,), jnp.float32),
        mesh=_mesh(),
        compiler_params=_sc_params(),
        scratch_types=[
            pltpu.VMEM((npp * n,), jnp.int32),
            pltpu.VMEM((fpt * n,), jnp.float32),
            pltpu.VMEM((ch,), jnp.int32),
            pltpu.VMEM((ch,), jnp.int32),
            pltpu.VMEM((ch,), jnp.float32),
            pltpu.VMEM((ch,), jnp.float32),
            pltpu.SemaphoreType.DMA,
            pltpu.SemaphoreType.DMA,
        ],
    )
    def agg_kernel(hP_hbm, pk_hbm, nrm_hbm, zero_hbm, out_hbm,
                   h_v, acc, pk0, pk1, nrm0, nrm1, sem0, sem1):
        wid = _wid()
        pltpu.sync_copy(hP_hbm.at[pl.ds(wid * (npp * n), npp * n)], h_v)
        pltpu.sync_copy(zero_hbm, acc)
        bufs = ((pk0, nrm0, sem0), (pk1, nrm1, sem1))

        def start(g, p):
            off = g * ch
            pv, nv, sem = bufs[p]
            pltpu.async_copy(pk_hbm.at[pl.ds(off, ch)], pv, sem)
            pltpu.async_copy(nrm_hbm.at[pl.ds(off, ch)], nv, sem)

        def wait(g, p):
            off = g * ch
            pv, nv, sem = bufs[p]
            pltpu.make_async_copy(pk_hbm.at[pl.ds(off, ch)], pv, sem).wait()
            pltpu.make_async_copy(nrm_hbm.at[pl.ds(off, ch)], nv, sem).wait()

        def compute(p):
            pv, nv, _ = bufs[p]

            @plsc.parallel_loop(0, ch, step=L, unroll=8)
            def body(i):
                pk = pv[pl.ds(i, L)]
                nrm = nv[pl.ds(i, L)]
                s = lax.bitwise_and(pk, 0xFFFF)
                d = lax.shift_right_logical(pk, 16)
                for pp in range(npp):
                    si = s + (pp * n) if pp else s
                    w = plsc.load_gather(h_v, [si])
                    lo = plsc.bitcast(lax.shift_left(w, 16), jnp.float32)
                    hi = plsc.bitcast(lax.bitwise_and(w, jnp.int32(-65536)),
                                      jnp.float32)
                    dlo = d + (pp * n) if pp else d
                    plsc.addupdate_scatter(acc, [dlo], nrm * lo)
                    plsc.addupdate_scatter(acc, [d + ((npp + pp) * n)], nrm * hi)

        start(0, 0)

        def outer(g2, _):
            g0 = g2 * 2
            start(g0 + 1, 1)
            wait(g0, 0)
            compute(0)

            @pl.when(g2 + 1 < nch // 2)
            def _prefetch():
                start(g0 + 2, 0)

            wait(g0 + 1, 1)
            compute(1)
            return 0

        lax.fori_loop(0, nch // 2, outer, 0)
        pltpu.sync_copy(acc.at[pl.ds(0, npp * n)],
                        out_hbm.at[pl.ds((npp * wid) * n, npp * n)])
        pltpu.sync_copy(acc.at[pl.ds(npp * n, npp * n)],
                        out_hbm.at[pl.ds((fh + npp * wid) * n, npp * n)])

    return agg_kernel


def _make_agg2_kernel(n, e, f):
    nchunks = NW // f
    ept = e // nchunks
    ch = 4000

    @functools.partial(
        pl.kernel,
        out_type=jax.ShapeDtypeStruct((NW * n,), jnp.float32),
        mesh=_mesh(),
        compiler_params=_sc_params(),
        scratch_types=[
            pltpu.VMEM((n,), jnp.float32),
            pltpu.VMEM((n,), jnp.float32),
            pltpu.VMEM((ch,), jnp.int32),
            pltpu.VMEM((ch,), jnp.float32),
        ],
    )
    def agg2_kernel(hT_hbm, pk_hbm, nrm_hbm, zero_hbm, out_hbm,
                    h_v, acc, pk_v, nrm_v):
        wid = _wid()
        ff = lax.rem(wid, f)
        cc = lax.div(wid, f)
        base = cc * ept
        pltpu.sync_copy(hT_hbm.at[pl.ds(ff * n, n)], h_v)
        pltpu.sync_copy(zero_hbm.at[pl.ds(0, n)], acc)

        def chunk(g, _):
            off = base + g * ch
            pltpu.sync_copy(pk_hbm.at[pl.ds(off, ch)], pk_v)
            pltpu.sync_copy(nrm_hbm.at[pl.ds(off, ch)], nrm_v)

            @plsc.parallel_loop(0, ch, step=L, unroll=8)
            def body(i):
                pk = pk_v[pl.ds(i, L)]
                nv = nrm_v[pl.ds(i, L)]
                s = lax.bitwise_and(pk, 0xFFFF)
                d = lax.shift_right_logical(pk, 16)
                hv = plsc.load_gather(h_v, [s])
                plsc.addupdate_scatter(acc, [d], nv * hv)

            return 0

        lax.fori_loop(0, ept // ch, chunk, 0)
        pltpu.sync_copy(acc, out_hbm.at[pl.ds(wid * n, n)])

    return agg2_kernel


def _degfin_body(p_ref, dis_ref, dis2_ref):
    deg = jnp.sum(p_ref[...], axis=0) + 1.0
    dis = jnp.where(deg > 0, lax.rsqrt(jnp.maximum(deg, 1e-12)), 0.0)
    dis_ref[...] = dis
    dis2_ref[...] = dis * dis


def _pack_pairs(hT):
    fh = hT.shape[0] // 2
    hb = hT.astype(jnp.bfloat16)
    u = lax.bitcast_convert_type(hb, jnp.uint16).astype(jnp.uint32)
    w = lax.bitwise_or(u[:fh], lax.shift_left(u[fh:], jnp.uint32(16)))
    return lax.bitcast_convert_type(w, jnp.int32)


def _mm1_body(w_ref, x_ref, o_ref, p_ref):
    hT = lax.dot_general(
        w_ref[...], x_ref[...], (([0], [1]), ([], [])),
        preferred_element_type=jnp.float32)
    o_ref[...] = hT
    p_ref[...] = _pack_pairs(hT)


def _layer_body(agg_ref, h_ref, dis2_ref, b_ref, w_ref, o_ref, p_ref):
    z = jnp.maximum(agg_ref[...] + dis2_ref[...] * h_ref[...] + b_ref[...], 0.0)
    hT = lax.dot_general(
        w_ref[...], z, (([0], [0]), ([], [])),
        preferred_element_type=jnp.float32)
    o_ref[...] = hT
    p_ref[...] = _pack_pairs(hT)


def _layer3_body(agg_ref, h_ref, dis2_ref, b_ref, w3_ref, wlin_ref, o_ref):
    z = jnp.maximum(agg_ref[...] + dis2_ref[...] * h_ref[...] + b_ref[...], 0.0)
    w3p = lax.dot_general(
        w3_ref[...], wlin_ref[...], (([1], [0]), ([], [])),
        preferred_element_type=jnp.float32)
    o_ref[...] = lax.dot_general(
        w3p, z, (([0], [0]), ([], [])),
        preferred_element_type=jnp.float32)


def _final_body(p_ref, h3_ref, dis2_ref, batch_ref, b3_ref, wlin_ref, blin_ref,
                o_ref, *, n, g):
    f = h3_ref.shape[0]
    agg3 = jnp.sum(p_ref[...].reshape(NW // f, f, n), axis=0)
    z3 = agg3 + dis2_ref[...] * h3_ref[...]
    gi = lax.broadcasted_iota(jnp.int32, (g, n), 0)
    m = (gi == batch_ref[...][None, :]).astype(jnp.float32)
    sums = lax.dot_general(m, z3, (([1], [1]), ([], [])),
                           preferred_element_type=jnp.float32)
    counts = lax.dot_general(m, jnp.ones((1, n), jnp.float32),
                             (([1], [1]), ([], [])),
                             preferred_element_type=jnp.float32)
    const = lax.dot_general(b3_ref[...], wlin_ref[...], (([1], [0]), ([], [])),
                            preferred_element_type=jnp.float32)
    o_ref[...] = sums / jnp.maximum(counts, 1.0) + const + blin_ref[...][None, :]


def kernel(x, edge_index, edge_weight, batch, W1, b1, W2, b2, W3, b3, Wlin, blin):
    n, d = x.shape
    e = edge_index.shape[1]
    h = W1.shape[1]
    c = Wlin.shape[1]
    g = 64
    fpt = h // NW

    src = edge_index[0].astype(jnp.int32)
    dst = edge_index[1].astype(jnp.int32)
    zeros_flat = jnp.zeros((fpt * n,), jnp.float32)

    deg_k = _make_deg_kernel(n, e)
    norm_k = _make_norm_kernel(n, e)
    agg_k = _make_agg_kernel(n, e, h)
    agg2_k = _make_agg2_kernel(n, e, c)

    partials = deg_k(dst, edge_weight, zeros_flat).reshape(NW, n)
    dis, dis2 = pl.pallas_call(
        _degfin_body,
        out_shape=(jax.ShapeDtypeStruct((n,), jnp.float32),
                   jax.ShapeDtypeStruct((n,), jnp.float32)),
    )(partials)
    norm, pk = norm_k(src, dst, edge_weight, dis)

    dis2r = dis2[None, :]
    h1T, h1P = pl.pallas_call(
        _mm1_body,
        out_shape=(jax.ShapeDtypeStruct((h, n), jnp.float32),
                   jax.ShapeDtypeStruct((h // 2, n), jnp.int32)))(W1, x)
    a1T = agg_k(h1P.reshape(-1), pk, norm, zeros_flat).reshape(h, n)
    h2T, h2P = pl.pallas_call(
        _layer_body,
        out_shape=(jax.ShapeDtypeStruct((h, n), jnp.float32),
                   jax.ShapeDtypeStruct((h // 2, n), jnp.int32)))(
        a1T, h1T, dis2r, b1[:, None], W2)
    a2T = agg_k(h2P.reshape(-1), pk, norm, zeros_flat).reshape(h, n)
    h3T = pl.pallas_call(
        _layer3_body, out_shape=jax.ShapeDtypeStruct((c, n), jnp.float32))(
        a2T, h2T, dis2r, b2[:, None], W3, Wlin)
    p3 = agg2_k(h3T.reshape(-1), pk, norm, zeros_flat).reshape(NW, n)
    out = pl.pallas_call(
        functools.partial(_final_body, n=n, g=g),
        out_shape=jax.ShapeDtypeStruct((g, c), jnp.float32),
    )(p3, h3T, dis2r, batch.astype(jnp.int32), b3[None, :], Wlin, blin)
    return out


# speedup vs baseline: 1.0123x; 1.0119x over previous
import functools

import jax
import jax.numpy as jnp
from jax import lax
from jax.experimental import pallas as pl
from jax.experimental.pallas import tpu as pltpu
from jax.experimental.pallas import tpu_sc as plsc

NC = 2
NS = 16
NW = NC * NS
L = 16


def _mesh():
    return plsc.VectorSubcoreMesh(
        core_axis_name="c", subcore_axis_name="s", num_cores=NC, num_subcores=NS
    )


def _sc_params():
    return pltpu.CompilerParams(needs_layout_passes=False)


def _wid():
    return lax.axis_index("s") * NC + lax.axis_index("c")


def _make_deg_kernel(n, e):
    ept = e // NW
    ch = 2000

    @functools.partial(
        pl.kernel,
        out_type=jax.ShapeDtypeStruct((NW * n,), jnp.float32),
        mesh=_mesh(),
        compiler_params=_sc_params(),
        scratch_types=[
            pltpu.VMEM((ch,), jnp.int32),
            pltpu.VMEM((ch,), jnp.float32),
            pltpu.VMEM((n,), jnp.float32),
        ],
    )
    def deg_kernel(dst_hbm, w_hbm, zero_hbm, out_hbm, dst_v, w_v, acc):
        wid = _wid()
        base = wid * ept
        pltpu.sync_copy(zero_hbm.at[pl.ds(0, n)], acc)

        def chunk(g, _):
            off = base + g * ch
            pltpu.sync_copy(dst_hbm.at[pl.ds(off, ch)], dst_v)
            pltpu.sync_copy(w_hbm.at[pl.ds(off, ch)], w_v)

            @plsc.parallel_loop(0, ch, step=L, unroll=8)
            def body(j):
                d = dst_v[pl.ds(j, L)]
                v = w_v[pl.ds(j, L)]
                plsc.addupdate_scatter(acc, [d], v)

            return 0

        lax.fori_loop(0, ept // ch, chunk, 0)
        pltpu.sync_copy(acc, out_hbm.at[pl.ds(wid * n, n)])

    return deg_kernel


def _make_norm_kernel(n, e):
    ept = e // NW
    ch = 2000

    @functools.partial(
        pl.kernel,
        out_type=(jax.ShapeDtypeStruct((e,), jnp.float32),
                  jax.ShapeDtypeStruct((e,), jnp.int32)),
        mesh=_mesh(),
        compiler_params=_sc_params(),
        scratch_types=[
            pltpu.VMEM((n,), jnp.float32),
            pltpu.VMEM((ch,), jnp.int32),
            pltpu.VMEM((ch,), jnp.int32),
            pltpu.VMEM((ch,), jnp.float32),
            pltpu.VMEM((ch,), jnp.float32),
            pltpu.VMEM((ch,), jnp.int32),
        ],
    )
    def norm_kernel(src_hbm, dst_hbm, w_hbm, dis_hbm, out_hbm, pk_hbm,
                    dis_v, src_v, dst_v, w_v, nrm_v, pk_v):
        wid = _wid()
        base = wid * ept
        pltpu.sync_copy(dis_hbm, dis_v)

        def chunk(g, _):
            off = base + g * ch
            pltpu.sync_copy(src_hbm.at[pl.ds(off, ch)], src_v)
            pltpu.sync_copy(dst_hbm.at[pl.ds(off, ch)], dst_v)
            pltpu.sync_copy(w_hbm.at[pl.ds(off, ch)], w_v)

            @plsc.parallel_loop(0, ch, step=L, unroll=8)
            def body(j):
                s = src_v[pl.ds(j, L)]
                d = dst_v[pl.ds(j, L)]
                wv = w_v[pl.ds(j, L)]
                ds_ = plsc.load_gather(dis_v, [s])
                dd_ = plsc.load_gather(dis_v, [d])
                nrm_v[pl.ds(j, L)] = ds_ * wv * dd_
                pk_v[pl.ds(j, L)] = lax.bitwise_or(s, lax.shift_left(d, 16))

            pltpu.sync_copy(nrm_v, out_hbm.at[pl.ds(off, ch)])
            pltpu.sync_copy(pk_v, pk_hbm.at[pl.ds(off, ch)])
            return 0

        lax.fori_loop(0, ept // ch, chunk, 0)

    return norm_kernel


def _make_agg_kernel(n, e, f):
    fpt = f // NW
    npp = fpt // 2
    fh = f // 2
    ch = 8000
    nch = e // ch
    assert e % ch == 0 and nch % 2 == 0 and ch % L == 0

    @functools.partial(
        pl.kernel,
        out_type=jax.ShapeDtypeStruct((f * n,), jnp.float32),
        mesh=_mesh(),
        compiler_params=_sc_params(),
        scratch_types=[
            pltpu.VMEM((npp * n,), jnp.int32),
            pltpu.VMEM((fpt * n,), jnp.float32),
            pltpu.VMEM((ch,), jnp.int32),
            pltpu.VMEM((ch,), jnp.int32),
            pltpu.VMEM((ch,), jnp.float32),
            pltpu.VMEM((ch,), jnp.float32),
            pltpu.SemaphoreType.DMA,
            pltpu.SemaphoreType.DMA,
        ],
    )
    def agg_kernel(hP_hbm, pk_hbm, nrm_hbm, zero_hbm, out_hbm,
                   h_v, acc, pk0, pk1, nrm0, nrm1, sem0, sem1):
        wid = _wid()
        pltpu.sync_copy(hP_hbm.at[pl.ds(wid * (npp * n), npp * n)], h_v)
        pltpu.sync_copy(zero_hbm, acc)
        bufs = ((pk0, nrm0, sem0), (pk1, nrm1, sem1))

        def start(g, p):
            off = g * ch
            pv, nv, sem = bufs[p]
            pltpu.async_copy(pk_hbm.at[pl.ds(off, ch)], pv, sem)
            pltpu.async_copy(nrm_hbm.at[pl.ds(off, ch)], nv, sem)

        def wait(g, p):
            off = g * ch
            pv, nv, sem = bufs[p]
            pltpu.make_async_copy(pk_hbm.at[pl.ds(off, ch)], pv, sem).wait()
            pltpu.make_async_copy(nrm_hbm.at[pl.ds(off, ch)], nv, sem).wait()

        def compute(p):
            pv, nv, _ = bufs[p]

            @plsc.parallel_loop(0, ch, step=L, unroll=8)
            def body(i):
                pk = pv[pl.ds(i, L)]
                nrm = nv[pl.ds(i, L)]
                s = lax.bitwise_and(pk, 0xFFFF)
                d = lax.shift_right_logical(pk, 16)
                for pp in range(npp):
                    si = s + (pp * n) if pp else s
                    w = plsc.load_gather(h_v, [si])
                    lo = plsc.bitcast(lax.shift_left(w, 16), jnp.float32)
                    hi = plsc.bitcast(lax.bitwise_and(w, jnp.int32(-65536)),
                                      jnp.float32)
                    dlo = d + (pp * n) if pp else d
                    plsc.addupdate_scatter(acc, [dlo], nrm * lo)
                    plsc.addupdate_scatter(acc, [d + ((npp + pp) * n)], nrm * hi)

        start(0, 0)

        def outer(g2, _):
            g0 = g2 * 2
            start(g0 + 1, 1)
            wait(g0, 0)
            compute(0)

            @pl.when(g2 + 1 < nch // 2)
            def _prefetch():
                start(g0 + 2, 0)

            wait(g0 + 1, 1)
            compute(1)
            return 0

        lax.fori_loop(0, nch // 2, outer, 0)
        pltpu.sync_copy(acc.at[pl.ds(0, npp * n)],
                        out_hbm.at[pl.ds((npp * wid) * n, npp * n)])
        pltpu.sync_copy(acc.at[pl.ds(npp * n, npp * n)],
                        out_hbm.at[pl.ds((fh + npp * wid) * n, npp * n)])

    return agg_kernel


def _make_agg2_kernel(n, e, f):
    nchunks = NW // f
    ept = e // nchunks
    ch = 4000

    @functools.partial(
        pl.kernel,
        out_type=jax.ShapeDtypeStruct((NW * n,), jnp.float32),
        mesh=_mesh(),
        compiler_params=_sc_params(),
        scratch_types=[
            pltpu.VMEM((n,), jnp.float32),
            pltpu.VMEM((n,), jnp.float32),
            pltpu.VMEM((ch,), jnp.int32),
            pltpu.VMEM((ch,), jnp.float32),
        ],
    )
    def agg2_kernel(hT_hbm, pk_hbm, nrm_hbm, zero_hbm, out_hbm,
                    h_v, acc, pk_v, nrm_v):
        wid = _wid()
        ff = lax.rem(wid, f)
        cc = lax.div(wid, f)
        base = cc * ept
        pltpu.sync_copy(hT_hbm.at[pl.ds(ff * n, n)], h_v)
        pltpu.sync_copy(zero_hbm.at[pl.ds(0, n)], acc)

        def chunk(g, _):
            off = base + g * ch
            pltpu.sync_copy(pk_hbm.at[pl.ds(off, ch)], pk_v)
            pltpu.sync_copy(nrm_hbm.at[pl.ds(off, ch)], nrm_v)

            @plsc.parallel_loop(0, ch, step=L, unroll=8)
            def body(i):
                pk = pk_v[pl.ds(i, L)]
                nv = nrm_v[pl.ds(i, L)]
                s = lax.bitwise_and(pk, 0xFFFF)
                d = lax.shift_right_logical(pk, 16)
                hv = plsc.load_gather(h_v, [s])
                plsc.addupdate_scatter(acc, [d], nv * hv)

            return 0

        lax.fori_loop(0, ept // ch, chunk, 0)
        pltpu.sync_copy(acc, out_hbm.at[pl.ds(wid * n, n)])

    return agg2_kernel


def _degfin_body(p_ref, dis_ref, dis2_ref):
    deg = jnp.sum(p_ref[...], axis=0) + 1.0
    dis = jnp.where(deg > 0, lax.rsqrt(jnp.maximum(deg, 1e-12)), 0.0)
    dis_ref[...] = dis
    dis2_ref[...] = dis * dis


def _pack_pairs(hT):
    fh = hT.shape[0] // 2
    hb = hT.astype(jnp.bfloat16)
    u = lax.bitcast_convert_type(hb, jnp.uint16).astype(jnp.uint32)
    w = lax.bitwise_or(u[:fh], lax.shift_left(u[fh:], jnp.uint32(16)))
    return lax.bitcast_convert_type(w, jnp.int32)


def _unpack_pairs(wp):
    lo = lax.bitcast_convert_type(lax.shift_left(wp, 16), jnp.float32)
    hi = lax.bitcast_convert_type(
        lax.bitwise_and(wp, jnp.int32(-65536)), jnp.float32)
    return jnp.concatenate([lo, hi], axis=0)


def _mm1_body(w_ref, x_ref, p_ref):
    hT = lax.dot_general(
        w_ref[...], x_ref[...], (([0], [1]), ([], [])),
        preferred_element_type=jnp.float32)
    p_ref[...] = _pack_pairs(hT)


def _layer_body(agg_ref, hp_ref, dis2_ref, b_ref, w_ref, p_ref):
    h = _unpack_pairs(hp_ref[...])
    z = jnp.maximum(agg_ref[...] + dis2_ref[...] * h + b_ref[...], 0.0)
    hT = lax.dot_general(
        w_ref[...], z, (([0], [0]), ([], [])),
        preferred_element_type=jnp.float32)
    p_ref[...] = _pack_pairs(hT)


def _layer3_body(agg_ref, hp_ref, dis2_ref, b_ref, w3_ref, wlin_ref, o_ref):
    h = _unpack_pairs(hp_ref[...])
    z = jnp.maximum(agg_ref[...] + dis2_ref[...] * h + b_ref[...], 0.0)
    w3p = lax.dot_general(
        w3_ref[...], wlin_ref[...], (([1], [0]), ([], [])),
        preferred_element_type=jnp.float32)
    o_ref[...] = lax.dot_general(
        w3p, z, (([0], [0]), ([], [])),
        preferred_element_type=jnp.float32)


def _final_body(p_ref, h3_ref, dis2_ref, batch_ref, b3_ref, wlin_ref, blin_ref,
                o_ref, *, n, g):
    f = h3_ref.shape[0]
    agg3 = jnp.sum(p_ref[...].reshape(NW // f, f, n), axis=0)
    z3 = agg3 + dis2_ref[...] * h3_ref[...]
    gi = lax.broadcasted_iota(jnp.int32, (g, n), 0)
    m = (gi == batch_ref[...][None, :]).astype(jnp.float32)
    sums = lax.dot_general(m, z3, (([1], [1]), ([], [])),
                           preferred_element_type=jnp.float32)
    counts = lax.dot_general(m, jnp.ones((1, n), jnp.float32),
                             (([1], [1]), ([], [])),
                             preferred_element_type=jnp.float32)
    const = lax.dot_general(b3_ref[...], wlin_ref[...], (([1], [0]), ([], [])),
                            preferred_element_type=jnp.float32)
    o_ref[...] = sums / jnp.maximum(counts, 1.0) + const + blin_ref[...][None, :]


def kernel(x, edge_index, edge_weight, batch, W1, b1, W2, b2, W3, b3, Wlin, blin):
    n, d = x.shape
    e = edge_index.shape[1]
    h = W1.shape[1]
    c = Wlin.shape[1]
    g = 64
    fpt = h // NW

    src = edge_index[0].astype(jnp.int32)
    dst = edge_index[1].astype(jnp.int32)
    zeros_flat = jnp.zeros((fpt * n,), jnp.float32)

    deg_k = _make_deg_kernel(n, e)
    norm_k = _make_norm_kernel(n, e)
    agg_k = _make_agg_kernel(n, e, h)
    agg2_k = _make_agg2_kernel(n, e, c)

    partials = deg_k(dst, edge_weight, zeros_flat).reshape(NW, n)
    dis, dis2 = pl.pallas_call(
        _degfin_body,
        out_shape=(jax.ShapeDtypeStruct((n,), jnp.float32),
                   jax.ShapeDtypeStruct((n,), jnp.float32)),
    )(partials)
    norm, pk = norm_k(src, dst, edge_weight, dis)

    dis2r = dis2[None, :]
    h1P = pl.pallas_call(
        _mm1_body,
        out_shape=jax.ShapeDtypeStruct((h // 2, n), jnp.int32))(W1, x)
    a1T = agg_k(h1P.reshape(-1), pk, norm, zeros_flat).reshape(h, n)
    h2P = pl.pallas_call(
        _layer_body,
        out_shape=jax.ShapeDtypeStruct((h // 2, n), jnp.int32))(
        a1T, h1P, dis2r, b1[:, None], W2)
    a2T = agg_k(h2P.reshape(-1), pk, norm, zeros_flat).reshape(h, n)
    h3T = pl.pallas_call(
        _layer3_body, out_shape=jax.ShapeDtypeStruct((c, n), jnp.float32))(
        a2T, h2P, dis2r, b2[:, None], W3, Wlin)
    p3 = agg2_k(h3T.reshape(-1), pk, norm, zeros_flat).reshape(NW, n)
    out = pl.pallas_call(
        functools.partial(_final_body, n=n, g=g),
        out_shape=jax.ShapeDtypeStruct((g, c), jnp.float32),
    )(p3, h3T, dis2r, batch.astype(jnp.int32), b3[None, :], Wlin, blin)
    return out
